# baseline (speedup 1.0000x reference)
import functools

import jax
import jax.numpy as jnp
from jax import lax
from jax.experimental import pallas as pl
from jax.experimental.pallas import tpu as pltpu
from jax.experimental.pallas import tpu_sc as plsc

VOCAB = 1000000
EMBED = 64
SEQ = 200
BATCH = 4096

NC = 2
NS = 16
NW = NC * NS

TOTAL = BATCH * SEQ
ROWS_PER_W = TOTAL // NW
SEQS_PER_CHUNK = 4
CHUNK = SEQS_PER_CHUNK * SEQ
NCHUNK = ROWS_PER_W // CHUNK
NDMA = 8
ROWS_PER_DMA = CHUNK // NDMA


def _embed_kernel(idx_hbm, tok_hbm, pos_hbm, out_hbm, idx_v, rows_v, pos_v, sem):
    wid = lax.axis_index("s") * NC + lax.axis_index("c")

    pltpu.sync_copy(pos_hbm, pos_v)

    def chunk_body(c, carry):
        base_i = wid * (ROWS_PER_W // ROWS_PER_DMA) + c * NDMA
        base_o = wid * ROWS_PER_W + c * CHUNK
        pltpu.sync_copy(idx_hbm.at[pl.ds(base_i, NDMA)], idx_v)
        copies = [
            pltpu.make_async_copy(
                tok_hbm.at[idx_v.at[j]],
                rows_v.at[pl.ds(j * ROWS_PER_DMA, ROWS_PER_DMA)],
                sem,
            )
            for j in range(NDMA)
        ]
        for cp in copies:
            cp.start()
        for cp in copies:
            cp.wait()

        def pos_body(p, carry2):
            for s in range(SEQS_PER_CHUNK):
                for g in range(EMBED // 16):
                    sl = pl.ds(g * 16, 16)
                    rows_v[s * SEQ + p, sl] = rows_v[s * SEQ + p, sl] + pos_v[p, sl]
            return carry2

        lax.fori_loop(0, SEQ, pos_body, 0)
        pltpu.sync_copy(rows_v, out_hbm.at[pl.ds(base_o, CHUNK)])
        return carry

    lax.fori_loop(0, NCHUNK, chunk_body, 0)


@jax.jit
def _embed(idx2d, token_table, position_table):
    mesh = plsc.VectorSubcoreMesh(core_axis_name="c", subcore_axis_name="s")
    return pl.kernel(
        _embed_kernel,
        mesh=mesh,
        out_type=jax.ShapeDtypeStruct((TOTAL, EMBED), jnp.float32),
        scratch_types=[
            pltpu.VMEM((NDMA, ROWS_PER_DMA), jnp.int32),
            pltpu.VMEM((CHUNK, EMBED), jnp.float32),
            pltpu.VMEM((SEQ, EMBED), jnp.float32),
            pltpu.SemaphoreType.DMA,
        ],
        compiler_params=pltpu.CompilerParams(use_tc_tiling_on_sc=False),
    )(idx2d, token_table, position_table)


def kernel(x, token_table, position_table):
    idx2d = x.reshape(TOTAL // ROWS_PER_DMA, ROWS_PER_DMA)
    out = _embed(idx2d, token_table, position_table)
    return out.reshape(BATCH, SEQ, EMBED)


# speedup vs baseline: 1.0932x; 1.0932x over previous
import jax
import jax.numpy as jnp
from jax import lax
from jax.experimental import pallas as pl
from jax.experimental.pallas import tpu as pltpu
from jax.experimental.pallas import tpu_sc as plsc

VOCAB = 1000000
EMBED = 64
SEQ = 200
BATCH = 4096

NC = 2
NS = 16
NW = NC * NS

TOTAL = BATCH * SEQ
ROWS_PER_W = TOTAL // NW
SEQS_PER_CHUNK = 4
CHUNK = SEQS_PER_CHUNK * SEQ
NCHUNK = ROWS_PER_W // CHUNK
NDMA = 8
ROWS_PER_DMA = CHUNK // NDMA


def _embed_kernel(idx_hbm, tok_hbm, pos_hbm, out_hbm,
                  idx0, idx1, rows0, rows1, pos_v,
                  sem_g0, sem_g1, sem_w0, sem_w1):
    wid = lax.axis_index("s") * NC + lax.axis_index("c")
    row0 = wid * ROWS_PER_W
    irow0 = wid * (ROWS_PER_W // ROWS_PER_DMA)
    idx_v = (idx0, idx1)
    rows_v = (rows0, rows1)
    sem_g = (sem_g0, sem_g1)
    sem_w = (sem_w0, sem_w1)

    pltpu.sync_copy(pos_hbm, pos_v)

    def gather_copies(b):
        return [
            pltpu.make_async_copy(
                tok_hbm.at[idx_v[b].at[j]],
                rows_v[b].at[pl.ds(j * ROWS_PER_DMA, ROWS_PER_DMA)],
                sem_g[b],
            )
            for j in range(NDMA)
        ]

    def copy_idx(b, c):
        pltpu.sync_copy(idx_hbm.at[pl.ds(irow0 + c * NDMA, NDMA)], idx_v[b])

    def wb_copy(b, c):
        return pltpu.make_async_copy(
            rows_v[b], out_hbm.at[pl.ds(row0 + c * CHUNK, CHUNK)], sem_w[b])

    def pos_add(b):
        def body(p, carry):
            for g in range(EMBED // 16):
                sl = pl.ds(g * 16, 16)
                pv = pos_v[p, sl]
                for s in range(SEQS_PER_CHUNK):
                    r = s * SEQ + p
                    rows_v[b][r, sl] = rows_v[b][r, sl] + pv
            return carry

        lax.fori_loop(0, SEQ, body, 0)

    copy_idx(0, 0)
    for cp in gather_copies(0):
        cp.start()

    def service(b, c, prep_next):
        if prep_next:
            copy_idx(1 - b, c + 1)
            for cp in gather_copies(1 - b):
                cp.start()
        for cp in gather_copies(b):
            cp.wait()
        pos_add(b)
        wb_copy(b, c).start()

    service(0, 0, True)

    def pair_body(g2, carry):
        for b in range(2):
            c = 1 + 2 * g2 + b
            bb = (1 + b) % 2
            wb_copy(1 - bb, c - 1).wait()
            service(bb, c, True)
        return carry

    lax.fori_loop(0, (NCHUNK - 2) // 2, pair_body, 0)

    wb_copy(0, NCHUNK - 2).wait()
    service(1, NCHUNK - 1, False)
    wb_copy(1, NCHUNK - 1).wait()


@jax.jit
def _embed(idx2d, token_table, position_table):
    mesh = plsc.VectorSubcoreMesh(core_axis_name="c", subcore_axis_name="s")
    return pl.kernel(
        _embed_kernel,
        mesh=mesh,
        out_type=jax.ShapeDtypeStruct((TOTAL, EMBED), jnp.float32),
        scratch_types=[
            pltpu.VMEM((NDMA, ROWS_PER_DMA), jnp.int32),
            pltpu.VMEM((NDMA, ROWS_PER_DMA), jnp.int32),
            pltpu.VMEM((CHUNK, EMBED), jnp.float32),
            pltpu.VMEM((CHUNK, EMBED), jnp.float32),
            pltpu.VMEM((SEQ, EMBED), jnp.float32),
            pltpu.SemaphoreType.DMA,
            pltpu.SemaphoreType.DMA,
            pltpu.SemaphoreType.DMA,
            pltpu.SemaphoreType.DMA,
        ],
        compiler_params=pltpu.CompilerParams(use_tc_tiling_on_sc=False),
    )(idx2d, token_table, position_table)


def kernel(x, token_table, position_table):
    idx2d = x.reshape(TOTAL // ROWS_PER_DMA, ROWS_PER_DMA)
    out = _embed(idx2d, token_table, position_table)
    return out.reshape(BATCH, SEQ, EMBED)
